# baseline (speedup 1.0000x reference)
import functools

import jax
import jax.numpy as jnp
from jax import lax
from jax.experimental import pallas as pl
from jax.experimental.pallas import tpu as pltpu
from jax.experimental.pallas import tpu_sc as plsc

D_MODEL = 768
MAXLEN = 30
N_SEG = 2
VOCAB = 4
N_ROWS = VOCAB * N_SEG * MAXLEN
LN_EPS = 1e-5


def _table_body(tok_ref, seg_ref, pos_ref, w_ref, b_ref, out_ref):
    def onehot(div, mod, n):
        r = lax.broadcasted_iota(jnp.int32, (N_ROWS, n), 0)
        c = lax.broadcasted_iota(jnp.int32, (N_ROWS, n), 1)
        return ((r // div) % mod == c).astype(jnp.float32)

    ov = onehot(N_SEG * MAXLEN, VOCAB, VOCAB)
    os_ = onehot(MAXLEN, N_SEG, N_SEG)
    ot = onehot(1, MAXLEN, MAXLEN)
    hi = lax.Precision.HIGHEST
    pre = (
        jnp.dot(ov, tok_ref[...], preferred_element_type=jnp.float32, precision=hi)
        + jnp.dot(os_, seg_ref[...], preferred_element_type=jnp.float32, precision=hi)
        + jnp.dot(ot, pos_ref[...], preferred_element_type=jnp.float32, precision=hi)
    )
    mu = jnp.mean(pre, axis=1, keepdims=True)
    var = jnp.mean(jnp.square(pre - mu), axis=1, keepdims=True)
    out_ref[...] = (pre - mu) * lax.rsqrt(var + LN_EPS) * w_ref[...] + b_ref[...]


N_REP = 64


def _build_table(tok_table, seg_table, pos_embed, ln_w, ln_b):
    return pl.pallas_call(
        _table_body,
        grid=(N_REP,),
        in_specs=[pl.BlockSpec(s, lambda r: (0,) * len(s)) for s in
                  [(VOCAB, D_MODEL), (N_SEG, D_MODEL), (MAXLEN, D_MODEL),
                   (1, D_MODEL), (1, D_MODEL)]],
        out_specs=pl.BlockSpec((N_ROWS, D_MODEL), lambda r: (r, 0)),
        out_shape=jax.ShapeDtypeStruct((N_REP * N_ROWS, D_MODEL), jnp.float32),
    )(tok_table, seg_table, pos_embed, ln_w.reshape(1, -1), ln_b.reshape(1, -1))


def _build_idx(x, seq, rows_per_worker):
    b = x.shape[0]
    blk = 2048

    def body(x_ref, seq_ref, out_ref):
        t = lax.broadcasted_iota(jnp.int32, x_ref.shape, 1)
        row = lax.broadcasted_iota(jnp.int32, x_ref.shape, 0) + pl.program_id(0) * blk
        rep_off = ((row // rows_per_worker) * 2 + (t % 2)) * N_ROWS
        out_ref[...] = (x_ref[...] * N_SEG + seq_ref[...]) * MAXLEN + t + rep_off

    return pl.pallas_call(
        body,
        grid=(b // blk,),
        in_specs=[
            pl.BlockSpec((blk, MAXLEN), lambda i: (i, 0)),
            pl.BlockSpec((blk, MAXLEN), lambda i: (i, 0)),
        ],
        out_specs=pl.BlockSpec((blk, MAXLEN), lambda i: (i, 0)),
        out_shape=jax.ShapeDtypeStruct((b, MAXLEN), jnp.int32),
    )(x, seq)


def _make_sc_gather(n_tokens):
    info = plsc.get_sparse_core_info()
    nc, ns = info.num_cores, info.num_subcores
    nw = nc * ns
    assert n_tokens % nw == 0
    t_per_w = n_tokens // nw
    chunk = 40
    nbuf = 3
    assert t_per_w % (chunk * nbuf) == 0
    n_chunks = t_per_w // chunk

    mesh = plsc.VectorSubcoreMesh(core_axis_name="c", subcore_axis_name="s")

    @functools.partial(
        pl.kernel,
        mesh=mesh,
        out_type=jax.ShapeDtypeStruct((n_tokens, D_MODEL), jnp.float32),
        scratch_types=[
            pltpu.VMEM((t_per_w,), jnp.int32),
            *[pltpu.VMEM((chunk, D_MODEL), jnp.float32) for _ in range(nbuf)],
            *[pltpu.SemaphoreType.DMA for _ in range(2 * nbuf)],
        ],
    )
    def gather_kernel(idx_hbm, table_hbm, out_hbm, idx_v, *bufs_sems):
        rows = bufs_sems[:nbuf]
        gsems = bufs_sems[nbuf : 2 * nbuf]
        ssems = bufs_sems[2 * nbuf :]
        wid = lax.axis_index("s") * nc + lax.axis_index("c")
        base = wid * t_per_w
        pltpu.sync_copy(idx_hbm.at[pl.ds(base, t_per_w)], idx_v)

        def start_gather(b, ci):
            pltpu.async_copy(
                table_hbm.at[idx_v.at[pl.ds(ci * chunk, chunk)]], rows[b], gsems[b]
            )

        def wait_gather(b):
            pltpu.make_async_copy(
                table_hbm.at[idx_v.at[pl.ds(0, chunk)]], rows[b], gsems[b]
            ).wait()

        def start_store(b, ci):
            pltpu.async_copy(
                rows[b], out_hbm.at[pl.ds(base + ci * chunk, chunk)], ssems[b]
            )

        def wait_store(b):
            pltpu.make_async_copy(
                rows[b], out_hbm.at[pl.ds(base, chunk)], ssems[b]
            ).wait()

        for b in range(nbuf - 1):
            start_gather(b, b)

        def body(g, carry):
            for b in range(nbuf):
                ci = g * nbuf + b
                wait_gather(b)
                start_store(b, ci)
                bn = (b + nbuf - 1) % nbuf

                @pl.when(ci >= 1)
                def _():
                    wait_store(bn)

                @pl.when(ci + nbuf - 1 < n_chunks)
                def _():
                    start_gather(bn, ci + nbuf - 1)
            return carry

        lax.fori_loop(0, n_chunks // nbuf, body, 0)
        wait_store((n_chunks - 1) % nbuf)

    return gather_kernel


def kernel(x, seq, tok_table, seg_table, pos_embed, ln_w, ln_b):
    b = x.shape[0]
    n_tokens = b * MAXLEN
    table = _build_table(tok_table, seg_table, pos_embed, ln_w, ln_b)
    idx = _build_idx(x, seq, b // 32).reshape(n_tokens)
    out = _make_sc_gather(n_tokens)(idx, table)
    return out.reshape(b, MAXLEN, D_MODEL)


# speedup vs baseline: 1.0306x; 1.0306x over previous
import functools

import jax
import jax.numpy as jnp
from jax import lax
from jax.experimental import pallas as pl
from jax.experimental.pallas import tpu as pltpu
from jax.experimental.pallas import tpu_sc as plsc

D_MODEL = 768
MAXLEN = 30
N_SEG = 2
VOCAB = 4
N_ROWS = VOCAB * N_SEG * MAXLEN
LN_EPS = 1e-5


def _table_body(tok_ref, seg_ref, pos_ref, w_ref, b_ref, out_ref):
    def onehot(div, mod, n):
        r = lax.broadcasted_iota(jnp.int32, (N_ROWS, n), 0)
        c = lax.broadcasted_iota(jnp.int32, (N_ROWS, n), 1)
        return ((r // div) % mod == c).astype(jnp.float32)

    ov = onehot(N_SEG * MAXLEN, VOCAB, VOCAB)
    os_ = onehot(MAXLEN, N_SEG, N_SEG)
    ot = onehot(1, MAXLEN, MAXLEN)
    hi = lax.Precision.HIGHEST
    pre = (
        jnp.dot(ov, tok_ref[...], preferred_element_type=jnp.float32, precision=hi)
        + jnp.dot(os_, seg_ref[...], preferred_element_type=jnp.float32, precision=hi)
        + jnp.dot(ot, pos_ref[...], preferred_element_type=jnp.float32, precision=hi)
    )
    mu = jnp.mean(pre, axis=1, keepdims=True)
    var = jnp.mean(jnp.square(pre - mu), axis=1, keepdims=True)
    out_ref[...] = (pre - mu) * lax.rsqrt(var + LN_EPS) * w_ref[...] + b_ref[...]


N_REP = 32


def _build_table(tok_table, seg_table, pos_embed, ln_w, ln_b):
    return pl.pallas_call(
        _table_body,
        grid=(N_REP,),
        in_specs=[pl.BlockSpec(s, lambda r: (0,) * len(s)) for s in
                  [(VOCAB, D_MODEL), (N_SEG, D_MODEL), (MAXLEN, D_MODEL),
                   (1, D_MODEL), (1, D_MODEL)]],
        out_specs=pl.BlockSpec((N_ROWS, D_MODEL), lambda r: (r, 0)),
        out_shape=jax.ShapeDtypeStruct((N_REP * N_ROWS, D_MODEL), jnp.float32),
    )(tok_table, seg_table, pos_embed, ln_w.reshape(1, -1), ln_b.reshape(1, -1))


def _build_idx(x, seq, rows_per_worker):
    b = x.shape[0]
    blk = 2048

    def body(x_ref, seq_ref, out_ref):
        t = lax.broadcasted_iota(jnp.int32, x_ref.shape, 1)
        row = lax.broadcasted_iota(jnp.int32, x_ref.shape, 0) + pl.program_id(0) * blk
        rep_off = (row // rows_per_worker) * N_ROWS
        out_ref[...] = (x_ref[...] * N_SEG + seq_ref[...]) * MAXLEN + t + rep_off

    return pl.pallas_call(
        body,
        grid=(b // blk,),
        in_specs=[
            pl.BlockSpec((blk, MAXLEN), lambda i: (i, 0)),
            pl.BlockSpec((blk, MAXLEN), lambda i: (i, 0)),
        ],
        out_specs=pl.BlockSpec((blk, MAXLEN), lambda i: (i, 0)),
        out_shape=jax.ShapeDtypeStruct((b, MAXLEN), jnp.int32),
    )(x, seq)


def _make_sc_gather(n_tokens):
    info = plsc.get_sparse_core_info()
    nc, ns = info.num_cores, info.num_subcores
    nw = nc * ns
    assert n_tokens % nw == 0
    t_per_w = n_tokens // nw
    chunk = 40
    nbuf = 3
    assert t_per_w % (chunk * nbuf) == 0
    n_chunks = t_per_w // chunk

    mesh = plsc.VectorSubcoreMesh(core_axis_name="c", subcore_axis_name="s")

    @functools.partial(
        pl.kernel,
        mesh=mesh,
        out_type=jax.ShapeDtypeStruct((n_tokens, D_MODEL), jnp.float32),
        scratch_types=[
            pltpu.VMEM((t_per_w,), jnp.int32),
            *[pltpu.VMEM((chunk, D_MODEL), jnp.float32) for _ in range(nbuf)],
            *[pltpu.SemaphoreType.DMA for _ in range(2 * nbuf)],
        ],
    )
    def gather_kernel(idx_hbm, table_hbm, out_hbm, idx_v, *bufs_sems):
        rows = bufs_sems[:nbuf]
        gsems = bufs_sems[nbuf : 2 * nbuf]
        ssems = bufs_sems[2 * nbuf :]
        wid = lax.axis_index("s") * nc + lax.axis_index("c")
        base = wid * t_per_w
        pltpu.sync_copy(idx_hbm.at[pl.ds(base, t_per_w)], idx_v)

        def start_gather(b, ci):
            pltpu.async_copy(
                table_hbm.at[idx_v.at[pl.ds(ci * chunk, chunk)]], rows[b], gsems[b]
            )

        def wait_gather(b):
            pltpu.make_async_copy(
                table_hbm.at[idx_v.at[pl.ds(0, chunk)]], rows[b], gsems[b]
            ).wait()

        def start_store(b, ci):
            pltpu.async_copy(
                rows[b], out_hbm.at[pl.ds(base + ci * chunk, chunk)], ssems[b]
            )

        def wait_store(b):
            pltpu.make_async_copy(
                rows[b], out_hbm.at[pl.ds(base, chunk)], ssems[b]
            ).wait()

        for b in range(nbuf - 1):
            start_gather(b, b)

        def body(g, carry):
            for b in range(nbuf):
                ci = g * nbuf + b
                wait_gather(b)
                start_store(b, ci)
                bn = (b + nbuf - 1) % nbuf

                @pl.when(ci >= 1)
                def _():
                    wait_store(bn)

                @pl.when(ci + nbuf - 1 < n_chunks)
                def _():
                    start_gather(bn, ci + nbuf - 1)
            return carry

        lax.fori_loop(0, n_chunks // nbuf, body, 0)
        wait_store((n_chunks - 1) % nbuf)

    return gather_kernel


def kernel(x, seq, tok_table, seg_table, pos_embed, ln_w, ln_b):
    b = x.shape[0]
    n_tokens = b * MAXLEN
    table = _build_table(tok_table, seg_table, pos_embed, ln_w, ln_b)
    idx = _build_idx(x, seq, b // 32).reshape(n_tokens)
    out = _make_sc_gather(n_tokens)(idx, table)
    return out.reshape(b, MAXLEN, D_MODEL)


# speedup vs baseline: 1.0310x; 1.0004x over previous
import functools

import jax
import jax.numpy as jnp
from jax import lax
from jax.experimental import pallas as pl
from jax.experimental.pallas import tpu as pltpu
from jax.experimental.pallas import tpu_sc as plsc

D_MODEL = 768
MAXLEN = 30
N_SEG = 2
VOCAB = 4
N_ROWS = VOCAB * N_SEG * MAXLEN
LN_EPS = 1e-5


def _table_body(tok_ref, seg_ref, pos_ref, w_ref, b_ref, out_ref):
    def onehot(div, mod, n):
        r = lax.broadcasted_iota(jnp.int32, (N_ROWS, n), 0)
        c = lax.broadcasted_iota(jnp.int32, (N_ROWS, n), 1)
        return ((r // div) % mod == c).astype(jnp.float32)

    ov = onehot(N_SEG * MAXLEN, VOCAB, VOCAB)
    os_ = onehot(MAXLEN, N_SEG, N_SEG)
    ot = onehot(1, MAXLEN, MAXLEN)
    hi = lax.Precision.HIGHEST
    pre = (
        jnp.dot(ov, tok_ref[...], preferred_element_type=jnp.float32, precision=hi)
        + jnp.dot(os_, seg_ref[...], preferred_element_type=jnp.float32, precision=hi)
        + jnp.dot(ot, pos_ref[...], preferred_element_type=jnp.float32, precision=hi)
    )
    mu = jnp.mean(pre, axis=1, keepdims=True)
    var = jnp.mean(jnp.square(pre - mu), axis=1, keepdims=True)
    out_ref[...] = (pre - mu) * lax.rsqrt(var + LN_EPS) * w_ref[...] + b_ref[...]


N_REP = 32


def _build_table(tok_table, seg_table, pos_embed, ln_w, ln_b):
    return pl.pallas_call(
        _table_body,
        grid=(N_REP,),
        in_specs=[pl.BlockSpec(s, lambda r: (0,) * len(s)) for s in
                  [(VOCAB, D_MODEL), (N_SEG, D_MODEL), (MAXLEN, D_MODEL),
                   (1, D_MODEL), (1, D_MODEL)]],
        out_specs=pl.BlockSpec((N_ROWS, D_MODEL), lambda r: (r, 0)),
        out_shape=jax.ShapeDtypeStruct((N_REP * N_ROWS, D_MODEL), jnp.float32),
    )(tok_table, seg_table, pos_embed, ln_w.reshape(1, -1), ln_b.reshape(1, -1))


def _build_idx(x, seq, rows_per_worker):
    b = x.shape[0]
    blk = 2048

    def body(x_ref, seq_ref, out_ref):
        t = lax.broadcasted_iota(jnp.int32, x_ref.shape, 1)
        row = lax.broadcasted_iota(jnp.int32, x_ref.shape, 0) + pl.program_id(0) * blk
        rep_off = (row // rows_per_worker) * N_ROWS
        out_ref[...] = (x_ref[...] * N_SEG + seq_ref[...]) * MAXLEN + t + rep_off

    return pl.pallas_call(
        body,
        grid=(b // blk,),
        in_specs=[
            pl.BlockSpec((blk, MAXLEN), lambda i: (i, 0)),
            pl.BlockSpec((blk, MAXLEN), lambda i: (i, 0)),
        ],
        out_specs=pl.BlockSpec((blk, MAXLEN), lambda i: (i, 0)),
        out_shape=jax.ShapeDtypeStruct((b, MAXLEN), jnp.int32),
    )(x, seq)


def _make_sc_gather(n_tokens):
    info = plsc.get_sparse_core_info()
    nc, ns = info.num_cores, info.num_subcores
    nw = nc * ns
    assert n_tokens % nw == 0
    t_per_w = n_tokens // nw
    chunk = 48
    nbuf = 2
    assert t_per_w % (chunk * nbuf) == 0
    n_chunks = t_per_w // chunk

    mesh = plsc.VectorSubcoreMesh(core_axis_name="c", subcore_axis_name="s")

    @functools.partial(
        pl.kernel,
        mesh=mesh,
        out_type=jax.ShapeDtypeStruct((n_tokens, D_MODEL), jnp.float32),
        scratch_types=[
            pltpu.VMEM((t_per_w,), jnp.int32),
            *[pltpu.VMEM((chunk, D_MODEL), jnp.float32) for _ in range(nbuf)],
            *[pltpu.SemaphoreType.DMA for _ in range(2 * nbuf)],
        ],
    )
    def gather_kernel(idx_hbm, table_hbm, out_hbm, idx_v, *bufs_sems):
        rows = bufs_sems[:nbuf]
        gsems = bufs_sems[nbuf : 2 * nbuf]
        ssems = bufs_sems[2 * nbuf :]
        wid = lax.axis_index("s") * nc + lax.axis_index("c")
        base = wid * t_per_w
        pltpu.sync_copy(idx_hbm.at[pl.ds(base, t_per_w)], idx_v)

        def start_gather(b, ci):
            pltpu.async_copy(
                table_hbm.at[idx_v.at[pl.ds(ci * chunk, chunk)]], rows[b], gsems[b]
            )

        def wait_gather(b):
            pltpu.make_async_copy(
                table_hbm.at[idx_v.at[pl.ds(0, chunk)]], rows[b], gsems[b]
            ).wait()

        def start_store(b, ci):
            pltpu.async_copy(
                rows[b], out_hbm.at[pl.ds(base + ci * chunk, chunk)], ssems[b]
            )

        def wait_store(b):
            pltpu.make_async_copy(
                rows[b], out_hbm.at[pl.ds(base, chunk)], ssems[b]
            ).wait()

        for b in range(nbuf - 1):
            start_gather(b, b)

        def body(g, carry):
            for b in range(nbuf):
                ci = g * nbuf + b
                wait_gather(b)
                start_store(b, ci)
                bn = (b + nbuf - 1) % nbuf

                @pl.when(ci >= 1)
                def _():
                    wait_store(bn)

                @pl.when(ci + nbuf - 1 < n_chunks)
                def _():
                    start_gather(bn, ci + nbuf - 1)
            return carry

        lax.fori_loop(0, n_chunks // nbuf, body, 0)
        wait_store((n_chunks - 1) % nbuf)

    return gather_kernel


def kernel(x, seq, tok_table, seg_table, pos_embed, ln_w, ln_b):
    b = x.shape[0]
    n_tokens = b * MAXLEN
    table = _build_table(tok_table, seg_table, pos_embed, ln_w, ln_b)
    idx = _build_idx(x, seq, b // 32).reshape(n_tokens)
    out = _make_sc_gather(n_tokens)(idx, table)
    return out.reshape(b, MAXLEN, D_MODEL)


# speedup vs baseline: 1.0316x; 1.0006x over previous
import functools

import jax
import jax.numpy as jnp
from jax import lax
from jax.experimental import pallas as pl
from jax.experimental.pallas import tpu as pltpu
from jax.experimental.pallas import tpu_sc as plsc

D_MODEL = 768
MAXLEN = 30
N_SEG = 2
VOCAB = 4
N_ROWS = VOCAB * N_SEG * MAXLEN
LN_EPS = 1e-5


def _table_body(tok_ref, seg_ref, pos_ref, w_ref, b_ref, out_ref):
    def onehot(div, mod, n):
        r = lax.broadcasted_iota(jnp.int32, (N_ROWS, n), 0)
        c = lax.broadcasted_iota(jnp.int32, (N_ROWS, n), 1)
        return ((r // div) % mod == c).astype(jnp.float32)

    ov = onehot(N_SEG * MAXLEN, VOCAB, VOCAB)
    os_ = onehot(MAXLEN, N_SEG, N_SEG)
    ot = onehot(1, MAXLEN, MAXLEN)
    hi = lax.Precision.HIGHEST
    pre = (
        jnp.dot(ov, tok_ref[...], preferred_element_type=jnp.float32, precision=hi)
        + jnp.dot(os_, seg_ref[...], preferred_element_type=jnp.float32, precision=hi)
        + jnp.dot(ot, pos_ref[...], preferred_element_type=jnp.float32, precision=hi)
    )
    mu = jnp.mean(pre, axis=1, keepdims=True)
    var = jnp.mean(jnp.square(pre - mu), axis=1, keepdims=True)
    out_ref[...] = (pre - mu) * lax.rsqrt(var + LN_EPS) * w_ref[...] + b_ref[...]


N_REP = 32


def _build_table(tok_table, seg_table, pos_embed, ln_w, ln_b):
    return pl.pallas_call(
        _table_body,
        grid=(N_REP,),
        in_specs=[pl.BlockSpec(s, lambda r: (0,) * len(s)) for s in
                  [(VOCAB, D_MODEL), (N_SEG, D_MODEL), (MAXLEN, D_MODEL),
                   (1, D_MODEL), (1, D_MODEL)]],
        out_specs=pl.BlockSpec((N_ROWS, D_MODEL), lambda r: (r, 0)),
        out_shape=jax.ShapeDtypeStruct((N_REP * N_ROWS, D_MODEL), jnp.float32),
    )(tok_table, seg_table, pos_embed, ln_w.reshape(1, -1), ln_b.reshape(1, -1))


def _build_idx(x, seq, rows_per_worker):
    b = x.shape[0]
    blk = 2048

    def body(x_ref, seq_ref, out_ref):
        t = lax.broadcasted_iota(jnp.int32, x_ref.shape, 1)
        row = lax.broadcasted_iota(jnp.int32, x_ref.shape, 0) + pl.program_id(0) * blk
        rep_off = (row // rows_per_worker) * N_ROWS
        out_ref[...] = (x_ref[...] * N_SEG + seq_ref[...]) * MAXLEN + t + rep_off

    return pl.pallas_call(
        body,
        grid=(b // blk,),
        in_specs=[
            pl.BlockSpec((blk, MAXLEN), lambda i: (i, 0)),
            pl.BlockSpec((blk, MAXLEN), lambda i: (i, 0)),
        ],
        out_specs=pl.BlockSpec((blk, MAXLEN), lambda i: (i, 0)),
        out_shape=jax.ShapeDtypeStruct((b, MAXLEN), jnp.int32),
    )(x, seq)


def _make_sc_gather(n_tokens):
    info = plsc.get_sparse_core_info()
    nc, ns = info.num_cores, info.num_subcores
    nw = nc * ns
    assert n_tokens % nw == 0
    t_per_w = n_tokens // nw
    chunk = 48
    nbuf = 2
    assert t_per_w % (chunk * nbuf) == 0
    n_chunks = t_per_w // chunk

    mesh = plsc.VectorSubcoreMesh(core_axis_name="c", subcore_axis_name="s")

    @functools.partial(
        pl.kernel,
        mesh=mesh,
        out_type=jax.ShapeDtypeStruct((n_tokens, D_MODEL), jnp.float32),
        scratch_types=[
            pltpu.VMEM((t_per_w,), jnp.int32),
            *[pltpu.VMEM((chunk, D_MODEL), jnp.float32) for _ in range(nbuf)],
            *[pltpu.SemaphoreType.DMA for _ in range(2 * nbuf)],
        ],
    )
    def gather_kernel(idx_hbm, table_hbm, out_hbm, idx_v, *bufs_sems):
        rows = bufs_sems[:nbuf]
        gsems = bufs_sems[nbuf : 2 * nbuf]
        ssems = bufs_sems[2 * nbuf :]
        wid = lax.axis_index("s") * nc + lax.axis_index("c")
        base = wid * t_per_w
        pltpu.sync_copy(idx_hbm.at[pl.ds(base, t_per_w)], idx_v)

        def start_gather(b, ci):
            pltpu.async_copy(
                table_hbm.at[idx_v.at[pl.ds(ci * chunk, chunk)]], rows[b], gsems[b]
            )

        def wait_gather(b):
            pltpu.make_async_copy(
                table_hbm.at[idx_v.at[pl.ds(0, chunk)]], rows[b], gsems[b]
            ).wait()

        def start_store(b, ci):
            pltpu.async_copy(
                rows[b], out_hbm.at[pl.ds(base + ci * chunk, chunk)], ssems[b]
            )

        def wait_store(b):
            pltpu.make_async_copy(
                rows[b], out_hbm.at[pl.ds(base, chunk)], ssems[b]
            ).wait()

        for b in range(nbuf - 1):
            start_gather(b, b)

        def body(g, carry):
            for b in range(nbuf):
                ci = g * nbuf + b
                wait_gather(b)
                start_store(b, ci)
                bn = (b + nbuf - 1) % nbuf

                @pl.when(ci >= 1)
                def _():
                    wait_store(bn)

                @pl.when(ci + nbuf - 1 < n_chunks)
                def _():
                    start_gather(bn, ci + nbuf - 1)
            return carry

        lax.fori_loop(0, n_chunks // nbuf, body, 0)
        wait_store((n_chunks - 1) % nbuf)

    return gather_kernel


def kernel(x, seq, tok_table, seg_table, pos_embed, ln_w, ln_b):
    b = x.shape[0]
    n_tokens = b * MAXLEN
    info = plsc.get_sparse_core_info()
    nw = info.num_cores * info.num_subcores
    assert N_REP == nw and b % nw == 0
    table = _build_table(tok_table, seg_table, pos_embed, ln_w, ln_b)
    idx = _build_idx(x, seq, b // nw).reshape(n_tokens)
    out = _make_sc_gather(n_tokens)(idx, table)
    return out.reshape(b, MAXLEN, D_MODEL)
